# Initial kernel scaffold; baseline (speedup 1.0000x reference)
import jax
import jax.numpy as jnp
from jax.experimental import pallas as pl


def kernel(links, paths, sequences, link_capacity, bandwith, n_links, n_paths, p_W_ih, p_W_hh, p_b_ih, p_b_hh, l_W_ih, l_W_hh, l_b_ih, l_b_hh, dW1, db1, dW2, db2, dW3, db3, jW1, jb1, jW2, jb2, jW3, jb3):
    raise NotImplementedError("write your pallas kernel here")


# speedup vs baseline: 2.7489x; 2.7489x over previous
import functools

import jax
import jax.numpy as jnp
from jax import lax
from jax.experimental import pallas as pl
from jax.experimental.pallas import tpu as pltpu
from jax.experimental.pallas import tpu_sc as plsc

_INTERP = False

D = 512
DH = 3 * D
S = 8
PB = 2048
SLOTS = S * PB
NLP = 10240
LBLK = 2048
E = 16000
EP = 16384


def _sc_gather_x(table, gidx):
    mesh = plsc.VectorSubcoreMesh(core_axis_name="c", subcore_axis_name="s")

    @functools.partial(
        pl.kernel, mesh=mesh,
        out_type=jax.ShapeDtypeStruct((SLOTS, D), jnp.float32),
        scratch_types=[
            pltpu.VMEM((128,), jnp.int32),
            pltpu.VMEM((128, D), jnp.float32),
            pltpu.SemaphoreType.DMA,
        ],
    )
    def k(table_hbm, gidx_hbm, out_hbm, idx_v, rows_v, sem):
        wid = lax.axis_index("s") * 2 + lax.axis_index("c")
        for c in range(4):
            base = wid * 512 + c * 128
            pltpu.sync_copy(gidx_hbm.at[pl.ds(base, 128)], idx_v)
            pltpu.async_copy(table_hbm.at[idx_v], rows_v, sem).wait()
            pltpu.sync_copy(rows_v, out_hbm.at[pl.ds(base, 128)])

    return k(table, gidx)


def _sc_scatter_add(seqt, slot2_p, ladd_p, zrows):
    mesh = plsc.VectorSubcoreMesh(core_axis_name="c", subcore_axis_name="s")
    RPT = NLP // 16

    @functools.partial(
        pl.kernel, mesh=mesh,
        out_type=jax.ShapeDtypeStruct((4, NLP, 128), jnp.float32),
        scratch_types=[
            pltpu.VMEM((128,), jnp.int32),
            pltpu.VMEM((128,), jnp.int32),
            pltpu.VMEM((128, 128), jnp.float32),
            pltpu.VMEM_SHARED((NLP, 128), jnp.float32),
            pltpu.SemaphoreType.DMA,
        ],
    )
    def k(seqt_hbm, sg_hbm, ss_hbm, z_hbm, out_hbm, ig_v, is_v, rows_v, acc,
          sem):
        cid = lax.axis_index("c")
        sid = lax.axis_index("s")
        for r in range(2):
            kchunk = r * 2 + cid
            pltpu.sync_copy(z_hbm, acc.at[pl.ds(sid * RPT, RPT)])
            plsc.subcore_barrier()
            for sub in range(8):
                base = sid * 1024 + sub * 128
                pltpu.sync_copy(sg_hbm.at[pl.ds(base, 128)], ig_v)
                pltpu.sync_copy(ss_hbm.at[pl.ds(base, 128)], is_v)
                pltpu.async_copy(seqt_hbm.at[kchunk].at[ig_v], rows_v,
                                 sem).wait()
                pltpu.sync_copy(rows_v, acc.at[is_v], add=True)
            plsc.subcore_barrier()
            pltpu.sync_copy(acc.at[pl.ds(sid * RPT, RPT)],
                            out_hbm.at[kchunk].at[pl.ds(sid * RPT, RPT)])
            plsc.subcore_barrier()

    return k(seqt, slot2_p, ladd_p, zrows)


def _path_gru_body(l_ref, x_ref, m_ref, h0_ref, wih_ref, whh_ref, bih_ref,
                   bhh_ref, seq_ref, ph_ref, h_s):
    s = pl.program_id(0)

    @pl.when(s == 0)
    def _():
        h_s[...] = h0_ref[...]

    h = h_s[...]
    x = x_ref[0] * m_ref[0]
    gi = jnp.dot(x, wih_ref[...], preferred_element_type=jnp.float32) + bih_ref[...]
    gh = jnp.dot(h, whh_ref[...], preferred_element_type=jnp.float32) + bhh_ref[...]
    r = jax.nn.sigmoid(gi[:, :D] + gh[:, :D])
    z = jax.nn.sigmoid(gi[:, D:2 * D] + gh[:, D:2 * D])
    n = jnp.tanh(gi[:, 2 * D:] + r * gh[:, 2 * D:])
    hn = (1.0 - z) * n + z * h
    hn = jnp.where(s < l_ref[0, 0], hn, h)
    h_s[...] = hn
    for c in range(4):
        seq_ref[c, 0] = hn[:, c * 128:(c + 1) * 128]

    @pl.when(s == S - 1)
    def _():
        ph_ref[...] = hn


def _path_gru(x3, m3, h0, wih_t, whh_t, b_ih, b_hh, l11):
    return pl.pallas_call(
        _path_gru_body,
        grid=(S,),
        in_specs=[
            pl.BlockSpec(memory_space=pltpu.SMEM),
            pl.BlockSpec((1, PB, D), lambda s: (s, 0, 0)),
            pl.BlockSpec((1, PB, 1), lambda s: (s, 0, 0)),
            pl.BlockSpec((PB, D), lambda s: (0, 0)),
            pl.BlockSpec((D, DH), lambda s: (0, 0)),
            pl.BlockSpec((D, DH), lambda s: (0, 0)),
            pl.BlockSpec((1, DH), lambda s: (0, 0)),
            pl.BlockSpec((1, DH), lambda s: (0, 0)),
        ],
        out_specs=[
            pl.BlockSpec((4, 1, PB, 128), lambda s: (0, s, 0, 0)),
            pl.BlockSpec((PB, D), lambda s: (0, 0)),
        ],
        out_shape=[
            jax.ShapeDtypeStruct((4, S, PB, 128), jnp.float32),
            jax.ShapeDtypeStruct((PB, D), jnp.float32),
        ],
        scratch_shapes=[pltpu.VMEM((PB, D), jnp.float32)],
        interpret=_INTERP,
    )(l11, x3, m3, h0, wih_t, whh_t, b_ih, b_hh)


def _link_gru_body(agg_ref, h_ref, wih_ref, whh_ref, bih_ref, bhh_ref, out_ref):
    h = h_ref[...]
    w = wih_ref[...]
    gi = bih_ref[...].astype(jnp.float32)
    for k in range(4):
        gi = gi + jnp.dot(agg_ref[k], w[k * 128:(k + 1) * 128, :],
                          preferred_element_type=jnp.float32)
    gh = jnp.dot(h, whh_ref[...], preferred_element_type=jnp.float32) + bhh_ref[...]
    r = jax.nn.sigmoid(gi[:, :D] + gh[:, :D])
    z = jax.nn.sigmoid(gi[:, D:2 * D] + gh[:, D:2 * D])
    n = jnp.tanh(gi[:, 2 * D:] + r * gh[:, 2 * D:])
    out_ref[...] = (1.0 - z) * n + z * h


def _link_gru(agg4, h, wih_t, whh_t, b_ih, b_hh):
    nblk = NLP // LBLK
    return pl.pallas_call(
        _link_gru_body,
        grid=(nblk,),
        in_specs=[
            pl.BlockSpec((4, LBLK, 128), lambda i: (0, i, 0)),
            pl.BlockSpec((LBLK, D), lambda i: (i, 0)),
            pl.BlockSpec((D, DH), lambda i: (0, 0)),
            pl.BlockSpec((D, DH), lambda i: (0, 0)),
            pl.BlockSpec((1, DH), lambda i: (0, 0)),
            pl.BlockSpec((1, DH), lambda i: (0, 0)),
        ],
        out_specs=pl.BlockSpec((LBLK, D), lambda i: (i, 0)),
        out_shape=jax.ShapeDtypeStruct((NLP, D), jnp.float32),
        compiler_params=pltpu.CompilerParams(
            dimension_semantics=("arbitrary",)),
        interpret=_INTERP,
    )(agg4, h, wih_t, whh_t, b_ih, b_hh)


def _readout_body(h_ref, dw1_ref, db1_ref, dw2_ref, db2_ref, dw3_ref, db3_ref,
                  jw1_ref, jb1_ref, jw2_ref, jb2_ref, jw3_ref, jb3_ref,
                  d_ref, j_ref):
    h = h_ref[...]
    a = jax.nn.relu(jnp.dot(h, dw1_ref[...], preferred_element_type=jnp.float32) + db1_ref[...])
    a = jax.nn.relu(jnp.dot(a, dw2_ref[...], preferred_element_type=jnp.float32) + db2_ref[...])
    d_ref[...] = jnp.dot(a, dw3_ref[...], preferred_element_type=jnp.float32) + db3_ref[...]
    b = jax.nn.relu(jnp.dot(h, jw1_ref[...], preferred_element_type=jnp.float32) + jb1_ref[...])
    b = jax.nn.relu(jnp.dot(b, jw2_ref[...], preferred_element_type=jnp.float32) + jb2_ref[...])
    j_ref[...] = jnp.dot(b, jw3_ref[...], preferred_element_type=jnp.float32) + jb3_ref[...]


def _readout(h, dw1_t, db1, dw2_t, db2, dw3_t, db3, jw1_t, jb1, jw2_t, jb2,
             jw3_t, jb3):
    return pl.pallas_call(
        _readout_body,
        out_shape=[
            jax.ShapeDtypeStruct((PB, 128), jnp.float32),
            jax.ShapeDtypeStruct((PB, 128), jnp.float32),
        ],
        interpret=_INTERP,
    )(h, dw1_t, db1, dw2_t, db2, dw3_t, db3, jw1_t, jb1, jw2_t, jb2, jw3_t, jb3)


def kernel(links, paths, sequences, link_capacity, bandwith, n_links, n_paths,
           p_W_ih, p_W_hh, p_b_ih, p_b_hh, l_W_ih, l_W_hh, l_b_ih, l_b_hh,
           dW1, db1, dW2, db2, dW3, db3, jW1, jb1, jW2, jb2, jW3, jb3):
    links_flat = links[:, 0].astype(jnp.int32)
    p_ind = paths[:, 0].astype(jnp.int32)
    s_ind = sequences[:, 0].astype(jnp.int32)

    n_paths_s = bandwith.shape[0]
    n_links_s = link_capacity.shape[0]
    zero_p = (jnp.asarray(n_paths) - n_paths_s).astype(jnp.float32)
    zero_l = (jnp.asarray(n_links) - n_links_s).astype(jnp.float32)

    slot2 = s_ind * PB + p_ind
    eids = jnp.arange(E, dtype=jnp.int32)
    slot_src = jnp.full((SLOTS,), -1, jnp.int32).at[slot2].max(eids)
    valid = slot_src >= 0
    gidx = jnp.where(valid, links_flat[jnp.clip(slot_src, 0)], 0)
    xmask3 = valid.astype(jnp.float32).reshape(S, PB, 1)
    slot2_p = jnp.concatenate([slot2, jnp.zeros((EP - E,), jnp.int32)])
    ladd_p = jnp.concatenate(
        [links_flat, jnp.full((EP - E,), NLP - 8, jnp.int32)])
    l11 = (jnp.max(s_ind) + 1).reshape(1, 1).astype(jnp.int32)

    path_h = jnp.zeros((PB, D), jnp.float32)
    path_h = path_h.at[:n_paths_s, 0:1].set(bandwith) + zero_p
    link_h = jnp.zeros((NLP, D), jnp.float32)
    link_h = link_h.at[:n_links_s, 0:1].set(link_capacity) + zero_l

    p_wih_t = p_W_ih.T
    p_whh_t = p_W_hh.T
    l_wih_t = l_W_ih.T
    l_whh_t = l_W_hh.T
    p_bih = p_b_ih.reshape(1, DH)
    p_bhh = p_b_hh.reshape(1, DH)
    l_bih = l_b_ih.reshape(1, DH)
    l_bhh = l_b_hh.reshape(1, DH)
    dw1_t, dw2_t = dW1.T, dW2.T
    jw1_t, jw2_t = jW1.T, jW2.T
    dw3_t = jnp.zeros((128, 128), jnp.float32).at[:, 0:1].set(dW3.T)
    jw3_t = jnp.zeros((128, 128), jnp.float32).at[:, 0:1].set(jW3.T)
    db1r, db2r = db1.reshape(1, -1), db2.reshape(1, -1)
    jb1r, jb2r = jb1.reshape(1, -1), jb2.reshape(1, -1)
    db3r = jnp.zeros((1, 128), jnp.float32).at[0, 0].set(db3[0])
    jb3r = jnp.zeros((1, 128), jnp.float32).at[0, 0].set(jb3[0])

    T = 8

    zrows = jnp.zeros((NLP // 16, 128), jnp.float32)

    def body(t, carry):
        link_h, path_h = carry
        x = _sc_gather_x(link_h, gidx)
        seq4, path_h = _path_gru(
            x.reshape(S, PB, D), xmask3, path_h,
            p_wih_t, p_whh_t, p_bih, p_bhh, l11)
        agg4 = _sc_scatter_add(seq4.reshape(4, SLOTS, 128), slot2_p, ladd_p,
                               zrows)
        link_h = _link_gru(agg4, link_h, l_wih_t, l_whh_t, l_bih, l_bhh)
        return (link_h, path_h)

    link_h, path_h = lax.fori_loop(0, T, body, (link_h, path_h))

    dfull, jfull = _readout(path_h, dw1_t, db1r, dw2_t, db2r, dw3_t, db3r,
                            jw1_t, jb1r, jw2_t, jb2r, jw3_t, jb3r)
    return (dfull[:n_paths_s, 0:1], jfull[:n_paths_s, 0:1])
